# Initial kernel scaffold; baseline (speedup 1.0000x reference)
import jax
import jax.numpy as jnp
from jax.experimental import pallas as pl


def kernel(x, vision, attention_mask, Wk, Wv, Wu, bu):
    raise NotImplementedError("write your pallas kernel here")


# speedup vs baseline: 16.7617x; 16.7617x over previous
import functools

import jax
import jax.numpy as jnp
from jax.experimental import pallas as pl

_VALID_BIAS = -100.0


def _meta_adapter_body(x_ref, vis_ref, mask_ref, wut_ref, wkt_ref, wvt_ref,
                       bu_ref, out_ref, *, nvis):
    x = x_ref[...]
    vis = vis_ref[...]
    mask = mask_ref[...]
    dim = x.shape[1]
    vp, t = mask.shape

    dn = (((1,), (0,)), ((), ()))
    up = jax.lax.dot_general(vis, wut_ref[...], dn,
                             preferred_element_type=jnp.float32,
                             precision=jax.lax.Precision.HIGHEST)
    up = up + bu_ref[...]
    k = jax.lax.dot_general(up, wkt_ref[...], dn,
                            preferred_element_type=jnp.float32,
                            precision=jax.lax.Precision.HIGHEST)
    v = jax.lax.dot_general(up, wvt_ref[...], dn,
                            preferred_element_type=jnp.float32,
                            precision=jax.lax.Precision.HIGHEST)

    s = jax.lax.dot_general(k, x, (((1,), (1,)), ((), ())),
                            preferred_element_type=jnp.float32,
                            precision=jax.lax.Precision.HIGHEST)
    scale = jax.lax.rsqrt(jnp.float32(dim))
    rowid = jax.lax.broadcasted_iota(jnp.int32, (vp, t), 0)
    logits = s * scale + jnp.where(rowid < nvis, jnp.float32(_VALID_BIAS),
                                   jnp.float32(0.0))

    cnt = jnp.sum(mask, axis=0, keepdims=True)
    max_len = jnp.max(cnt)
    npad = max_len - cnt
    w = jnp.where(rowid == nvis, npad, mask)

    lm = jnp.where(w > 0, logits, -jnp.inf)
    m = jnp.max(lm)
    e = w * jnp.exp(lm - m)
    z = jnp.sum(e)
    p = e / z

    attn = jax.lax.dot_general(p, v, (((0,), (0,)), ((), ())),
                               preferred_element_type=jnp.float32,
                               precision=jax.lax.Precision.HIGHEST)
    out_ref[...] = attn + x


def kernel(x, vision, attention_mask, Wk, Wv, Wu, bu):
    b, t, dim = x.shape
    v = vision.shape[1]
    cv = vision.shape[2]
    vp = ((v + 1 + 7) // 8) * 8

    x2 = x.reshape(t, dim)
    vis = jnp.zeros((vp, cv), jnp.float32).at[:v, :].set(vision.reshape(v, cv))
    maskf = jnp.zeros((vp, t), jnp.float32).at[:v, :].set(
        attention_mask.reshape(v, t).astype(jnp.float32))

    out = pl.pallas_call(
        functools.partial(_meta_adapter_body, nvis=v),
        out_shape=jax.ShapeDtypeStruct((t, dim), jnp.float32),
    )(x2, vis, maskf, Wu.T, Wk.T, Wv.T, bu.reshape(1, dim))
    return out.reshape(b, t, dim)


# speedup vs baseline: 23.0957x; 1.3779x over previous
import functools

import jax
import jax.numpy as jnp
from jax.experimental import pallas as pl
from jax.experimental.pallas import tpu as pltpu

_VALID_BIAS = -100.0
_DN_NT = (((1,), (1,)), ((), ()))
_DN_TN = (((0,), (0,)), ((), ()))


def _body(x_ref, vis_ref, mask_ref, wu_ref, wk_ref, wv_ref, bu_ref,
          out_ref, k_ref, v_ref, s_ref, w_ref, *, nvis, tile):
    ph = pl.program_id(0)
    i = pl.program_id(1)
    vp, t = s_ref.shape
    dim = x_ref.shape[1]
    scale = jax.lax.rsqrt(jnp.float32(dim))

    @pl.when(jnp.logical_and(ph == 0, i == 0))
    def _setup():
        up = jax.lax.dot_general(vis_ref[...], wu_ref[...], _DN_NT,
                                 preferred_element_type=jnp.float32)
        up = up + bu_ref[...]
        k_ref[...] = jax.lax.dot_general(up, wk_ref[...], _DN_NT,
                                         preferred_element_type=jnp.float32)
        v_ref[...] = jax.lax.dot_general(up, wv_ref[...], _DN_NT,
                                         preferred_element_type=jnp.float32)
        mask = mask_ref[...]
        cnt = jnp.sum(mask, axis=0, keepdims=True)
        npad = jnp.max(cnt) - cnt
        rowid = jax.lax.broadcasted_iota(jnp.int32, (vp, t), 0)
        w_ref[...] = jnp.where(rowid == nvis, npad, mask)

    @pl.when(ph == 0)
    def _scores():
        s_ref[:, pl.ds(i * tile, tile)] = jax.lax.dot_general(
            k_ref[...], x_ref[...], _DN_NT, preferred_element_type=jnp.float32)

    @pl.when(jnp.logical_and(ph == 1, i == 0))
    def _normalize():
        w = w_ref[...]
        rowid = jax.lax.broadcasted_iota(jnp.int32, (vp, t), 0)
        bias = jnp.where(rowid < nvis, jnp.float32(_VALID_BIAS),
                         jnp.float32(0.0))
        lm = jnp.where(w > 0, s_ref[...] * scale + bias, -jnp.inf)
        e = w * jnp.exp(lm - jnp.max(lm))
        s_ref[...] = e / jnp.sum(e)

    @pl.when(ph == 1)
    def _combine():
        p = s_ref[:, pl.ds(i * tile, tile)]
        attn = jax.lax.dot_general(p, v_ref[...], _DN_TN,
                                   preferred_element_type=jnp.float32)
        out_ref[...] = attn + x_ref[...]


def kernel(x, vision, attention_mask, Wk, Wv, Wu, bu):
    b, t, dim = x.shape
    v = vision.shape[1]
    cv = vision.shape[2]
    vp = ((v + 1 + 7) // 8) * 8
    tile = 256
    nt = t // tile

    x2 = x.reshape(t, dim)
    vis = jnp.zeros((vp, cv), jnp.float32).at[:v, :].set(vision.reshape(v, cv))
    maskf = jnp.zeros((vp, t), jnp.float32).at[:v, :].set(
        attention_mask.reshape(v, t).astype(jnp.float32))

    full = lambda shape: pl.BlockSpec(shape, lambda ph, i: (0, 0))
    out = pl.pallas_call(
        functools.partial(_body, nvis=v, tile=tile),
        grid=(2, nt),
        in_specs=[
            pl.BlockSpec((tile, dim), lambda ph, i: (i, 0)),
            full((vp, cv)),
            full((vp, t)),
            full((dim, cv)),
            full((dim, dim)),
            full((dim, dim)),
            full((1, dim)),
        ],
        out_specs=pl.BlockSpec((tile, dim),
                               lambda ph, i: (jnp.where(ph == 0, 0, i), 0)),
        out_shape=jax.ShapeDtypeStruct((t, dim), jnp.float32),
        scratch_shapes=[
            pltpu.VMEM((vp, dim), jnp.float32),
            pltpu.VMEM((vp, dim), jnp.float32),
            pltpu.VMEM((vp, t), jnp.float32),
            pltpu.VMEM((vp, t), jnp.float32),
        ],
    )(x2, vis, maskf, Wu, Wk, Wv, bu.reshape(1, dim))
    return out.reshape(b, t, dim)


# speedup vs baseline: 33.0706x; 1.4319x over previous
import functools

import jax
import jax.numpy as jnp
from jax.experimental import pallas as pl
from jax.experimental.pallas import tpu as pltpu

_VALID_BIAS = -100.0
_DN_NT = (((1,), (1,)), ((), ()))
_DN_TN = (((0,), (0,)), ((), ()))


def _body(x_ref, vis_ref, mask_ref, wu_ref, wk_ref, wv_ref, bu_ref,
          out_ref, k_ref, v_ref, s_ref, p_ref, w_ref, xs_ref, *, nvis, tile):
    ph = pl.program_id(0)
    i = pl.program_id(1)
    vp, t = s_ref.shape
    dim = x_ref.shape[1]
    scale = jax.lax.rsqrt(jnp.float32(dim))

    @pl.when(jnp.logical_and(ph == 0, i == 0))
    def _setup():
        vis = jnp.concatenate(
            [vis_ref[...], jnp.zeros((vp - nvis, vis_ref.shape[1]),
                                     jnp.float32)], axis=0)
        up = jax.lax.dot_general(vis, wu_ref[...], _DN_NT,
                                 preferred_element_type=jnp.float32)
        up = up + bu_ref[...]
        k_ref[...] = jax.lax.dot_general(up, wk_ref[...], _DN_NT,
                                         preferred_element_type=jnp.float32)
        v_ref[...] = jax.lax.dot_general(up, wv_ref[...], _DN_NT,
                                         preferred_element_type=jnp.float32
                                         ).astype(jnp.bfloat16)
        mask = jnp.concatenate(
            [mask_ref[...].astype(jnp.float32),
             jnp.zeros((vp - nvis, t), jnp.float32)], axis=0)
        cnt = jnp.sum(mask, axis=0, keepdims=True)
        npad = jnp.max(cnt) - cnt
        rowid = jax.lax.broadcasted_iota(jnp.int32, (vp, t), 0)
        w_ref[...] = jnp.where(rowid == nvis, npad, mask)

    @pl.when(ph == 0)
    def _scores():
        xt = x_ref[...]
        xs_ref[pl.ds(i * tile, tile), :] = xt
        s_ref[:, pl.ds(i * tile, tile)] = jax.lax.dot_general(
            k_ref[...], xt, _DN_NT, preferred_element_type=jnp.float32)

    @pl.when(jnp.logical_and(ph == 1, i == 0))
    def _normalize():
        w = w_ref[...]
        rowid = jax.lax.broadcasted_iota(jnp.int32, (vp, t), 0)
        bias = jnp.where(rowid < nvis, jnp.float32(_VALID_BIAS),
                         jnp.float32(0.0))
        lm = jnp.where(w > 0, s_ref[...] * scale + bias, -jnp.inf)
        e = w * jnp.exp(lm - jnp.max(lm))
        p_ref[...] = (e / jnp.sum(e)).astype(jnp.bfloat16)

    @pl.when(ph == 1)
    def _combine():
        p = p_ref[:, pl.ds(i * tile, tile)]
        attn = jax.lax.dot_general(p, v_ref[...], _DN_TN,
                                   preferred_element_type=jnp.float32)
        out_ref[...] = attn + xs_ref[pl.ds(i * tile, tile), :]


def kernel(x, vision, attention_mask, Wk, Wv, Wu, bu):
    b, t, dim = x.shape
    v = vision.shape[1]
    cv = vision.shape[2]
    vp = ((v + 1 + 7) // 8) * 8
    tile = 256
    nt = t // tile

    full = lambda shape: pl.BlockSpec(shape, lambda ph, i: (0, 0))
    out = pl.pallas_call(
        functools.partial(_body, nvis=v, tile=tile),
        grid=(2, nt),
        in_specs=[
            pl.BlockSpec((tile, dim),
                         lambda ph, i: (jnp.where(ph == 0, i, nt - 1), 0)),
            full((v, cv)),
            full((v, t)),
            full((dim, cv)),
            full((dim, dim)),
            full((dim, dim)),
            full((1, dim)),
        ],
        out_specs=pl.BlockSpec((tile, dim),
                               lambda ph, i: (jnp.where(ph == 0, 0, i), 0)),
        out_shape=jax.ShapeDtypeStruct((t, dim), jnp.float32),
        scratch_shapes=[
            pltpu.VMEM((vp, dim), jnp.float32),
            pltpu.VMEM((vp, dim), jnp.bfloat16),
            pltpu.VMEM((vp, t), jnp.float32),
            pltpu.VMEM((vp, t), jnp.bfloat16),
            pltpu.VMEM((vp, t), jnp.float32),
            pltpu.VMEM((t, dim), jnp.float32),
        ],
    )(x.reshape(t, dim), vision.reshape(v, cv),
      attention_mask.reshape(v, t), Wu, Wk, Wv, bu.reshape(1, dim))
    return out.reshape(b, t, dim)


# speedup vs baseline: 43.1634x; 1.3052x over previous
import functools

import jax
import jax.numpy as jnp
from jax.experimental import pallas as pl
from jax.experimental.pallas import tpu as pltpu

_VALID_BIAS = -100.0
_DN_NT = (((1,), (1,)), ((), ()))
_DN_TN = (((0,), (0,)), ((), ()))


def _body(x_ref, vis_ref, mask_ref, wu_ref, wk_ref, wv_ref, bu_ref,
          out_ref, k_ref, v_ref, s_ref, p_ref, w_ref, xs_ref, *, nvis, tile):
    ph = pl.program_id(0)
    i = pl.program_id(1)
    vp, t = s_ref.shape
    dim = x_ref.shape[1]
    scale = jax.lax.rsqrt(jnp.float32(dim))

    @pl.when(jnp.logical_and(ph == 0, i == 0))
    def _setup():
        vis = jnp.concatenate(
            [vis_ref[...], jnp.zeros((vp - nvis, vis_ref.shape[1]),
                                     jnp.float32)], axis=0)
        up = jax.lax.dot_general(vis, wu_ref[...], _DN_NT,
                                 preferred_element_type=jnp.float32)
        up = up + bu_ref[...]
        k_ref[...] = jax.lax.dot_general(up, wk_ref[...], _DN_NT,
                                         preferred_element_type=jnp.float32)
        v_ref[...] = jax.lax.dot_general(up, wv_ref[...], _DN_NT,
                                         preferred_element_type=jnp.float32
                                         ).astype(jnp.bfloat16)
        mask = jnp.concatenate(
            [mask_ref[...].astype(jnp.float32),
             jnp.zeros((vp - nvis, t), jnp.float32)], axis=0)
        cnt = jnp.sum(mask, axis=0, keepdims=True)
        npad = jnp.max(cnt) - cnt
        rowid = jax.lax.broadcasted_iota(jnp.int32, (vp, t), 0)
        w_ref[...] = jnp.where(rowid == nvis, npad, mask)

    @pl.when(ph == 0)
    def _scores():
        xt = x_ref[...]
        xs_ref[pl.ds(i * tile, tile), :] = xt
        s_ref[:, pl.ds(i * tile, tile)] = jax.lax.dot_general(
            k_ref[...], xt, _DN_NT, preferred_element_type=jnp.float32)

    @pl.when(jnp.logical_and(ph == 1, i == 0))
    def _normalize():
        w = w_ref[...]
        rowid = jax.lax.broadcasted_iota(jnp.int32, (vp, t), 0)
        bias = jnp.where(rowid < nvis, jnp.float32(_VALID_BIAS),
                         jnp.float32(0.0))
        lm = jnp.where(w > 0, s_ref[...] * scale + bias, -jnp.inf)
        e = w * jnp.exp(lm - jnp.max(lm))
        p_ref[...] = (e / jnp.sum(e)).astype(jnp.bfloat16)

    @pl.when(ph == 1)
    def _combine():
        p = p_ref[:, pl.ds(i * tile, tile)]
        attn = jax.lax.dot_general(p, v_ref[...], _DN_TN,
                                   preferred_element_type=jnp.float32)
        out_ref[...] = attn + xs_ref[pl.ds(i * tile, tile), :]


def kernel(x, vision, attention_mask, Wk, Wv, Wu, bu):
    b, t, dim = x.shape
    v = vision.shape[1]
    cv = vision.shape[2]
    vp = ((v + 1 + 7) // 8) * 8
    tile = 512
    nt = t // tile

    full = lambda shape: pl.BlockSpec(shape, lambda ph, i: (0, 0))
    out = pl.pallas_call(
        functools.partial(_body, nvis=v, tile=tile),
        grid=(2, nt),
        in_specs=[
            pl.BlockSpec((tile, dim),
                         lambda ph, i: (jnp.where(ph == 0, i, nt - 1), 0)),
            full((v, cv)),
            full((v, t)),
            full((dim, cv)),
            full((dim, dim)),
            full((dim, dim)),
            full((1, dim)),
        ],
        out_specs=pl.BlockSpec((tile, dim),
                               lambda ph, i: (jnp.where(ph == 0, 0, i), 0)),
        out_shape=jax.ShapeDtypeStruct((t, dim), jnp.float32),
        scratch_shapes=[
            pltpu.VMEM((vp, dim), jnp.float32),
            pltpu.VMEM((vp, dim), jnp.bfloat16),
            pltpu.VMEM((vp, t), jnp.float32),
            pltpu.VMEM((vp, t), jnp.bfloat16),
            pltpu.VMEM((vp, t), jnp.float32),
            pltpu.VMEM((t, dim), jnp.float32),
        ],
    )(x.reshape(t, dim), vision.reshape(v, cv),
      attention_mask.reshape(v, t), Wu, Wk, Wv, bu.reshape(1, dim))
    return out.reshape(b, t, dim)


# speedup vs baseline: 48.7206x; 1.1287x over previous
import functools

import jax
import jax.numpy as jnp
from jax.experimental import pallas as pl
from jax.experimental.pallas import tpu as pltpu

_VALID_BIAS = -100.0
_DN_NT = (((1,), (1,)), ((), ()))
_DN_TN = (((0,), (0,)), ((), ()))


def _body(x_ref, vis_ref, mask_ref, wu_ref, wk_ref, wv_ref, bu_ref,
          out_ref, k_ref, v_ref, s_ref, p_ref, w_ref, xs_ref, *, nvis, tile):
    ph = pl.program_id(0)
    i = pl.program_id(1)
    vp, t = s_ref.shape
    dim = x_ref.shape[1]
    scale = jax.lax.rsqrt(jnp.float32(dim))

    @pl.when(jnp.logical_and(ph == 0, i == 0))
    def _setup():
        vis = jnp.concatenate(
            [vis_ref[...], jnp.zeros((vp - nvis, vis_ref.shape[1]),
                                     jnp.float32)], axis=0)
        up = jax.lax.dot_general(vis, wu_ref[...], _DN_NT,
                                 preferred_element_type=jnp.float32)
        up = up + bu_ref[...]
        k_ref[...] = jax.lax.dot_general(up, wk_ref[...], _DN_NT,
                                         preferred_element_type=jnp.float32)
        v_ref[...] = jax.lax.dot_general(up, wv_ref[...], _DN_NT,
                                         preferred_element_type=jnp.float32
                                         ).astype(jnp.bfloat16)
        mask = jnp.concatenate(
            [mask_ref[...].astype(jnp.float32),
             jnp.zeros((vp - nvis, t), jnp.float32)], axis=0)
        cnt = jnp.sum(mask, axis=0, keepdims=True)
        npad = jnp.max(cnt) - cnt
        rowid = jax.lax.broadcasted_iota(jnp.int32, (vp, t), 0)
        w_ref[...] = jnp.where(rowid == nvis, npad, mask)

    @pl.when(ph == 0)
    def _scores():
        xt = x_ref[...]
        xs_ref[pl.ds(i * tile, tile), :] = xt
        s_ref[:, pl.ds(i * tile, tile)] = jax.lax.dot_general(
            k_ref[...], xt, _DN_NT, preferred_element_type=jnp.float32)

    @pl.when(jnp.logical_and(ph == 1, i == 0))
    def _normalize():
        w = w_ref[...]
        rowid = jax.lax.broadcasted_iota(jnp.int32, (vp, t), 0)
        bias = jnp.where(rowid < nvis, jnp.float32(_VALID_BIAS),
                         jnp.float32(0.0))
        lm = jnp.where(w > 0, s_ref[...] * scale + bias, -jnp.inf)
        e = w * jnp.exp(lm - jnp.max(lm))
        p_ref[...] = (e / jnp.sum(e)).astype(jnp.bfloat16)

    @pl.when(ph == 1)
    def _combine():
        p = p_ref[:, pl.ds(i * tile, tile)]
        attn = jax.lax.dot_general(p, v_ref[...], _DN_TN,
                                   preferred_element_type=jnp.float32)
        out_ref[...] = attn + xs_ref[pl.ds(i * tile, tile), :]


def kernel(x, vision, attention_mask, Wk, Wv, Wu, bu):
    b, t, dim = x.shape
    v = vision.shape[1]
    cv = vision.shape[2]
    vp = ((v + 1 + 7) // 8) * 8
    tile = 1024
    nt = t // tile

    full = lambda shape: pl.BlockSpec(shape, lambda ph, i: (0, 0))
    out = pl.pallas_call(
        functools.partial(_body, nvis=v, tile=tile),
        grid=(2, nt),
        in_specs=[
            pl.BlockSpec((tile, dim),
                         lambda ph, i: (jnp.where(ph == 0, i, nt - 1), 0)),
            full((v, cv)),
            full((v, t)),
            full((dim, cv)),
            full((dim, dim)),
            full((dim, dim)),
            full((1, dim)),
        ],
        out_specs=pl.BlockSpec((tile, dim),
                               lambda ph, i: (jnp.where(ph == 0, 0, i), 0)),
        out_shape=jax.ShapeDtypeStruct((t, dim), jnp.float32),
        scratch_shapes=[
            pltpu.VMEM((vp, dim), jnp.float32),
            pltpu.VMEM((vp, dim), jnp.bfloat16),
            pltpu.VMEM((vp, t), jnp.float32),
            pltpu.VMEM((vp, t), jnp.bfloat16),
            pltpu.VMEM((vp, t), jnp.float32),
            pltpu.VMEM((t, dim), jnp.float32),
        ],
    )(x.reshape(t, dim), vision.reshape(v, cv),
      attention_mask.reshape(v, t), Wu, Wk, Wv, bu.reshape(1, dim))
    return out.reshape(b, t, dim)
